# baseline (speedup 1.0000x reference)
import functools

import numpy as np
import jax
import jax.numpy as jnp
from jax import lax
from jax.experimental import pallas as pl
from jax.experimental.pallas import tpu as pltpu
from jax.experimental.pallas import tpu_sc as plsc

_D = 1024
_LM = 4096
_F = 64
_T = 576
_TP = _T // 9
_K = 12
_NSEL = _K + 1
_TEMP = 0.5
_LR = 0.1
_ALPHA = 0.8
_EPS = 1e-20
_BF = 8
_NSC = 16
_FTC = _F - _NSC
_HALF = _D // 2
_RCS = 48
_NCHS = (_T // 2) // _RCS


def _ema_matrix() -> np.ndarray:
    L = np.zeros((_F - 1, _F - 1), dtype=np.float64)
    for i in range(_F - 1):
        L[i, 0] = (1.0 - _ALPHA) ** i
        for j in range(1, i + 1):
            L[i, j] = _ALPHA * (1.0 - _ALPHA) ** (i - j)
    Dmat = np.zeros((_F - 1, _F), dtype=np.float64)
    for i in range(_F - 1):
        Dmat[i, i] = -1.0
        Dmat[i, i + 1] = 1.0
    M = np.zeros((_F, _F), dtype=np.float64)
    M[: _F - 1] = L @ Dmat
    return M.astype(np.float32)


def _pool_matrix() -> np.ndarray:
    P = np.zeros((_TP, _T), dtype=np.float32)
    for q in range(_TP):
        P[q, 9 * q : 9 * q + 9] = 1.0 / 9.0
    return P


_M_CONST = _ema_matrix()
_P_CONST = _pool_matrix()


def _fmean_tc_body(v_ref, out_ref):
    out_ref[...] = jnp.sum(v_ref[...], axis=1) * (1.0 / _T)


def _fmean_tc(video):
    return pl.pallas_call(
        _fmean_tc_body,
        grid=(_FTC // _BF,),
        in_specs=[pl.BlockSpec((_BF, _T, _D), lambda i: (i, 0, 0))],
        out_specs=pl.BlockSpec((_BF, _D), lambda i: (i, 0)),
        out_shape=jax.ShapeDtypeStruct((_FTC, _D), jnp.float32),
    )(video[:_FTC])


def _fmean_sc_body(vid_hbm, out_hbm, buf0, buf1, accv, sem0, sem1):
    cid = lax.axis_index("c")
    sid = lax.axis_index("s")
    wid = sid * 2 + cid
    f = wid // 2
    h = wid - 2 * f
    row0 = (_FTC + f) * _T + h * (_T // 2)
    bufs = (buf0, buf1)
    sems = (sem0, sem1)

    def _copy(k):
        return pltpu.make_async_copy(
            vid_hbm.at[pl.ds(row0 + k * _RCS, _RCS)],
            bufs[k % 2], sems[k % 2])

    _copy(0).start()
    accs = [jnp.zeros((16,), jnp.float32) for _ in range(_D // 16)]
    for k in range(_NCHS):
        if k + 1 < _NCHS:
            _copy(k + 1).start()
        _copy(k).wait()
        b = bufs[k % 2]
        for g in range(_D // 64):
            def body(r, carry, _b=b, _g=g):
                r8 = r * 8
                a = list(carry)
                for u in range(8):
                    for j in range(4):
                        a[j] = a[j] + _b[r8 + u, pl.ds(_g * 64 + j * 16, 16)]
                return tuple(a)
            res = lax.fori_loop(0, _RCS // 8, body,
                                tuple(accs[g * 4:(g + 1) * 4]))
            accs[g * 4:(g + 1) * 4] = list(res)
    for c in range(_D // 16):
        accv[pl.ds(c * 16, 16)] = accs[c] * (1.0 / _T)
    pltpu.sync_copy(accv, out_hbm.at[h, f])


def _fmean_sc(video2d):
    mesh = plsc.VectorSubcoreMesh(core_axis_name="c", subcore_axis_name="s")
    run = pl.kernel(
        _fmean_sc_body,
        mesh=mesh,
        out_type=jax.ShapeDtypeStruct((2, _NSC, _D), jnp.float32),
        scratch_types=[
            pltpu.VMEM((_RCS, _D), jnp.float32),
            pltpu.VMEM((_RCS, _D), jnp.float32),
            pltpu.VMEM((_D,), jnp.float32),
            pltpu.SemaphoreType.DMA,
            pltpu.SemaphoreType.DMA,
        ],
    )
    return run(video2d)


def _score_body(fmtc_ref, fmsc_ref, m_ref, wagg_ref, bagg_ref, lng_ref,
                lnb_ref, w1_ref, b1_ref, w2_ref, b2_ref, u_ref,
                idx_ref, wts_ref):
    fm_sc = fmsc_ref[0] + fmsc_ref[1]
    fm = jnp.concatenate([fmtc_ref[...], fm_sc], axis=0)
    mom = jnp.dot(m_ref[...], fm, preferred_element_type=jnp.float32)
    feats = jnp.dot(mom, wagg_ref[...],
                    preferred_element_type=jnp.float32) + bagg_ref[...]
    mu = jnp.mean(feats, axis=-1, keepdims=True)
    var = jnp.mean((feats - mu) ** 2, axis=-1, keepdims=True)
    h = (feats - mu) / jnp.sqrt(var + 1e-5) * lng_ref[...] + lnb_ref[...]
    h = jnp.dot(h, w1_ref[...], preferred_element_type=jnp.float32) + b1_ref[...]
    h = jax.nn.gelu(h)
    z = jnp.dot(h, w2_ref[...], preferred_element_type=jnp.float32) + b2_ref[...]
    g = -jnp.log(-jnp.log(u_ref[...] + _EPS) + _EPS)
    a = (z + g * _LR) / _TEMP
    a0 = a[:, 0:1]
    a1 = a[:, 1:2]
    mx = jnp.maximum(a0, a1)
    e0 = jnp.exp(a0 - mx)
    e1 = jnp.exp(a1 - mx)
    y = e1 / (e0 + e1)
    iota = jax.lax.broadcasted_iota(jnp.int32, (_F, 1), 0)
    ycur = jnp.where(iota < _F - 1, y, -jnp.inf)
    idx_ref[0] = 0
    vs = []
    for t in range(_K):
        m = jnp.max(ycur)
        first = jnp.min(jnp.where(ycur == m, iota, jnp.int32(2**30)))
        idx_ref[t + 1] = first + 1
        vs.append((1.0 - m) + m)
        ycur = jnp.where(iota == first, -jnp.inf, ycur)
    for t in range(_K + 1, 16):
        idx_ref[t] = 0
    s = 1.0
    for v in vs:
        s = s + v
    wts_ref[0] = 1.0 / s
    for t, v in enumerate(vs):
        wts_ref[t + 1] = v / s
    for t in range(_K + 1, 16):
        wts_ref[t] = 0.0


def _scores(fm_tc, fm_sc, W_agg, b_agg, ln_g, ln_b, W1, b1, W2, b2, U):
    Mc = jnp.asarray(_M_CONST)
    W2p = jnp.zeros((_D, 128), jnp.float32).at[:, :2].set(W2)
    b2p = jnp.zeros((1, 128), jnp.float32).at[0, :2].set(b2)
    Up = jnp.full((_F, 128), 0.5, jnp.float32).at[: _F - 1, :2].set(U[0])
    return pl.pallas_call(
        _score_body,
        in_specs=[pl.BlockSpec((_FTC, _D), lambda: (0, 0)),
                  pl.BlockSpec((2, _NSC, _D), lambda: (0, 0, 0)),
                  pl.BlockSpec((_F, _F), lambda: (0, 0)),
                  pl.BlockSpec((_D, _D), lambda: (0, 0)),
                  pl.BlockSpec((1, _D), lambda: (0, 0)),
                  pl.BlockSpec((1, _D), lambda: (0, 0)),
                  pl.BlockSpec((1, _D), lambda: (0, 0)),
                  pl.BlockSpec((_D, _D), lambda: (0, 0)),
                  pl.BlockSpec((1, _D), lambda: (0, 0)),
                  pl.BlockSpec((_D, 128), lambda: (0, 0)),
                  pl.BlockSpec((1, 128), lambda: (0, 0)),
                  pl.BlockSpec((_F, 128), lambda: (0, 0))],
        out_specs=[pl.BlockSpec(memory_space=pltpu.SMEM),
                   pl.BlockSpec(memory_space=pltpu.SMEM)],
        out_shape=[jax.ShapeDtypeStruct((16,), jnp.int32),
                   jax.ShapeDtypeStruct((16,), jnp.float32)],
    )(fm_tc, fm_sc, Mc, W_agg, b_agg.reshape(1, _D), ln_g.reshape(1, _D),
      ln_b.reshape(1, _D), W1, b1.reshape(1, _D), W2p, b2p, Up)


def _thumb_body(idx_ref, v_ref, pool_ref, wts_ref, wth_hbm, bth_ref,
                out_ref, acc_ref, wth_vmem, sem):
    i = pl.program_id(0)

    @pl.when(i == 0)
    def _():
        acc_ref[...] = jnp.zeros_like(acc_ref)
        pltpu.make_async_copy(wth_hbm, wth_vmem, sem).start()

    w = wts_ref[i]
    acc_ref[...] += w * jnp.dot(pool_ref[...], v_ref[0],
                                preferred_element_type=jnp.float32)

    @pl.when(i == _NSEL - 1)
    def _():
        pltpu.make_async_copy(wth_hbm, wth_vmem, sem).wait()
        out_ref[...] = jnp.dot(acc_ref[...], wth_vmem[...],
                               preferred_element_type=jnp.float32,
                               precision=jax.lax.Precision.HIGHEST) + bth_ref[...]


def _thumbnail(video, idx16, wts16, W_th, b_th):
    Pc = jnp.asarray(_P_CONST)
    grid_spec = pltpu.PrefetchScalarGridSpec(
        num_scalar_prefetch=1,
        grid=(_NSEL,),
        in_specs=[
            pl.BlockSpec((1, _T, _D), lambda i, idx_ref: (idx_ref[i], 0, 0)),
            pl.BlockSpec((_TP, _T), lambda i, idx_ref: (0, 0)),
            pl.BlockSpec(memory_space=pltpu.SMEM),
            pl.BlockSpec(memory_space=pl.ANY),
            pl.BlockSpec((1, _LM), lambda i, idx_ref: (0, 0)),
        ],
        out_specs=pl.BlockSpec((_TP, _LM), lambda i, idx_ref: (0, 0)),
        scratch_shapes=[pltpu.VMEM((_TP, _D), jnp.float32),
                        pltpu.VMEM((_D, _LM), jnp.float32),
                        pltpu.SemaphoreType.DMA],
    )
    return pl.pallas_call(
        _thumb_body,
        grid_spec=grid_spec,
        out_shape=jax.ShapeDtypeStruct((_TP, _LM), jnp.float32),
    )(idx16, video, Pc, wts16, W_th, b_th.reshape(1, _LM))


def kernel(video, U, W_agg, b_agg, ln_g, ln_b, W1, b1, W2, b2, W_th, b_th):
    fm_sc = _fmean_sc(video.reshape(_F * _T, _D))
    fm_tc = _fmean_tc(video)
    idx16, wts16 = _scores(fm_tc, fm_sc, W_agg, b_agg, ln_g, ln_b,
                           W1, b1, W2, b2, U)
    out = _thumbnail(video, idx16, wts16, W_th, b_th)
    return out[None]


# speedup vs baseline: 2.0342x; 2.0342x over previous
import functools

import numpy as np
import jax
import jax.numpy as jnp
from jax.experimental import pallas as pl
from jax.experimental.pallas import tpu as pltpu

_D = 1024
_LM = 4096
_F = 64
_T = 576
_TP = _T // 9
_K = 12
_NSEL = _K + 1
_TEMP = 0.5
_LR = 0.1
_ALPHA = 0.8
_EPS = 1e-20
_BF = 8


def _ema_matrix() -> np.ndarray:
    L = np.zeros((_F - 1, _F - 1), dtype=np.float64)
    for i in range(_F - 1):
        L[i, 0] = (1.0 - _ALPHA) ** i
        for j in range(1, i + 1):
            L[i, j] = _ALPHA * (1.0 - _ALPHA) ** (i - j)
    Dmat = np.zeros((_F - 1, _F), dtype=np.float64)
    for i in range(_F - 1):
        Dmat[i, i] = -1.0
        Dmat[i, i + 1] = 1.0
    M = np.zeros((_F, _F), dtype=np.float64)
    M[: _F - 1] = L @ Dmat
    return M.astype(np.float32)


def _pool_matrix() -> np.ndarray:
    P = np.zeros((_TP, _T), dtype=np.float32)
    for q in range(_TP):
        P[q, 9 * q : 9 * q + 9] = 1.0 / 9.0
    return P


_M_CONST = _ema_matrix()
_P_CONST = _pool_matrix()


def _fmean_score_body(v_ref, m_ref, wagg_ref, bagg_ref, lng_ref, lnb_ref,
                      w1_ref, b1_ref, w2_ref, b2_ref, u_ref,
                      idx_ref, wts_ref, fm_ref):
    i = pl.program_id(0)
    fm_ref[pl.ds(i * _BF, _BF), :] = (
        jnp.sum(v_ref[...], axis=1) * (1.0 / _T))

    @pl.when(i == _F // _BF - 1)
    def _():
        fm = fm_ref[...]
        mom = jnp.dot(m_ref[...], fm, preferred_element_type=jnp.float32)
        feats = jnp.dot(mom, wagg_ref[...],
                        preferred_element_type=jnp.float32) + bagg_ref[...]
        mu = jnp.mean(feats, axis=-1, keepdims=True)
        var = jnp.mean((feats - mu) ** 2, axis=-1, keepdims=True)
        h = (feats - mu) / jnp.sqrt(var + 1e-5) * lng_ref[...] + lnb_ref[...]
        h = jnp.dot(h, w1_ref[...], preferred_element_type=jnp.float32) + b1_ref[...]
        h = jax.nn.gelu(h)
        z = jnp.dot(h, w2_ref[...], preferred_element_type=jnp.float32) + b2_ref[...]
        g = -jnp.log(-jnp.log(u_ref[...] + _EPS) + _EPS)
        a = (z + g * _LR) / _TEMP
        a0 = a[:, 0:1]
        a1 = a[:, 1:2]
        mx = jnp.maximum(a0, a1)
        e0 = jnp.exp(a0 - mx)
        e1 = jnp.exp(a1 - mx)
        y = e1 / (e0 + e1)
        iota = jax.lax.broadcasted_iota(jnp.int32, (_F, 1), 0)
        ycur = jnp.where(iota < _F - 1, y, -jnp.inf)
        idx_ref[0] = 0
        vs = []
        for t in range(_K):
            m = jnp.max(ycur)
            first = jnp.min(jnp.where(ycur == m, iota, jnp.int32(2**30)))
            idx_ref[t + 1] = first + 1
            vs.append((1.0 - m) + m)
            ycur = jnp.where(iota == first, -jnp.inf, ycur)
        for t in range(_K + 1, 16):
            idx_ref[t] = 0
        s = 1.0
        for v in vs:
            s = s + v
        wts_ref[0] = 1.0 / s
        for t, v in enumerate(vs):
            wts_ref[t + 1] = v / s
        for t in range(_K + 1, 16):
            wts_ref[t] = 0.0


def _fmean_scores(video, W_agg, b_agg, ln_g, ln_b, W1, b1, W2, b2, U):
    Mc = jnp.asarray(_M_CONST)
    W2p = jnp.zeros((_D, 128), jnp.float32).at[:, :2].set(W2)
    b2p = jnp.zeros((1, 128), jnp.float32).at[0, :2].set(b2)
    Up = jnp.full((_F, 128), 0.5, jnp.float32).at[: _F - 1, :2].set(U[0])
    out = pl.pallas_call(
        _fmean_score_body,
        grid=(_F // _BF,),
        in_specs=[pl.BlockSpec((_BF, _T, _D), lambda i: (i, 0, 0)),
                  pl.BlockSpec((_F, _F), lambda i: (0, 0)),
                  pl.BlockSpec((_D, _D), lambda i: (0, 0)),
                  pl.BlockSpec((1, _D), lambda i: (0, 0)),
                  pl.BlockSpec((1, _D), lambda i: (0, 0)),
                  pl.BlockSpec((1, _D), lambda i: (0, 0)),
                  pl.BlockSpec((_D, _D), lambda i: (0, 0)),
                  pl.BlockSpec((1, _D), lambda i: (0, 0)),
                  pl.BlockSpec((_D, 128), lambda i: (0, 0)),
                  pl.BlockSpec((1, 128), lambda i: (0, 0)),
                  pl.BlockSpec((_F, 128), lambda i: (0, 0))],
        out_specs=[pl.BlockSpec(memory_space=pltpu.SMEM),
                   pl.BlockSpec(memory_space=pltpu.SMEM)],
        out_shape=[jax.ShapeDtypeStruct((16,), jnp.int32),
                   jax.ShapeDtypeStruct((16,), jnp.float32)],
        scratch_shapes=[pltpu.VMEM((_F, _D), jnp.float32)],
    )(video, Mc, W_agg, b_agg.reshape(1, _D), ln_g.reshape(1, _D),
      ln_b.reshape(1, _D), W1, b1.reshape(1, _D), W2p, b2p, Up)
    return out


def _thumb_body(idx_ref, v_ref, pool_ref, wts_ref, wth_hbm, bth_ref,
                out_ref, acc_ref, wth_vmem, sem):
    i = pl.program_id(0)

    @pl.when(i == 0)
    def _():
        acc_ref[...] = jnp.zeros_like(acc_ref)
        pltpu.make_async_copy(wth_hbm, wth_vmem, sem).start()

    w = wts_ref[i]
    acc_ref[...] += w * jnp.dot(pool_ref[...], v_ref[0],
                                preferred_element_type=jnp.float32)

    @pl.when(i == _NSEL - 1)
    def _():
        pltpu.make_async_copy(wth_hbm, wth_vmem, sem).wait()
        out_ref[...] = jnp.dot(acc_ref[...], wth_vmem[...],
                               preferred_element_type=jnp.float32) + bth_ref[...]


def _thumbnail(video, idx16, wts16, W_th, b_th):
    Pc = jnp.asarray(_P_CONST)
    grid_spec = pltpu.PrefetchScalarGridSpec(
        num_scalar_prefetch=1,
        grid=(_NSEL,),
        in_specs=[
            pl.BlockSpec((1, _T, _D), lambda i, idx_ref: (idx_ref[i], 0, 0)),
            pl.BlockSpec((_TP, _T), lambda i, idx_ref: (0, 0)),
            pl.BlockSpec(memory_space=pltpu.SMEM),
            pl.BlockSpec(memory_space=pl.ANY),
            pl.BlockSpec((1, _LM), lambda i, idx_ref: (0, 0)),
        ],
        out_specs=pl.BlockSpec((_TP, _LM), lambda i, idx_ref: (0, 0)),
        scratch_shapes=[pltpu.VMEM((_TP, _D), jnp.float32),
                        pltpu.VMEM((_D, _LM), jnp.float32),
                        pltpu.SemaphoreType.DMA],
    )
    return pl.pallas_call(
        _thumb_body,
        grid_spec=grid_spec,
        out_shape=jax.ShapeDtypeStruct((_TP, _LM), jnp.float32),
    )(idx16, video, Pc, wts16, W_th, b_th.reshape(1, _LM))


def kernel(video, U, W_agg, b_agg, ln_g, ln_b, W1, b1, W2, b2, W_th, b_th):
    idx16, wts16 = _fmean_scores(video, W_agg, b_agg, ln_g, ln_b,
                                 W1, b1, W2, b2, U)
    out = _thumbnail(video, idx16, wts16, W_th, b_th)
    return out[None]


# speedup vs baseline: 2.4556x; 1.2072x over previous
import functools

import numpy as np
import jax
import jax.numpy as jnp
from jax.experimental import pallas as pl
from jax.experimental.pallas import tpu as pltpu

_D = 1024
_LM = 4096
_F = 64
_T = 576
_TP = _T // 9
_K = 12
_NSEL = _K + 1
_TEMP = 0.5
_LR = 0.1
_ALPHA = 0.8
_EPS = 1e-20
_BF = 2


def _ema_matrix() -> np.ndarray:
    L = np.zeros((_F - 1, _F - 1), dtype=np.float64)
    for i in range(_F - 1):
        L[i, 0] = (1.0 - _ALPHA) ** i
        for j in range(1, i + 1):
            L[i, j] = _ALPHA * (1.0 - _ALPHA) ** (i - j)
    Dmat = np.zeros((_F - 1, _F), dtype=np.float64)
    for i in range(_F - 1):
        Dmat[i, i] = -1.0
        Dmat[i, i + 1] = 1.0
    M = np.zeros((_F, _F), dtype=np.float64)
    M[: _F - 1] = L @ Dmat
    return M.astype(np.float32)


def _pool_matrix() -> np.ndarray:
    P = np.zeros((_TP, _T), dtype=np.float32)
    for q in range(_TP):
        P[q, 9 * q : 9 * q + 9] = 1.0 / 9.0
    return P


_M_CONST = _ema_matrix()
_P_CONST = _pool_matrix()


def _body(v_ref, pool_ref, m_ref, wagg_ref, bagg_ref, lng_ref, lnb_ref,
          w1_ref, b1_ref, w2_ref, b2_ref, u_ref, wth_hbm, bth_ref,
          out_ref, pooled_ref, fm3_ref, wth_vmem, sem):
    i = pl.program_id(0)

    @pl.when(i == 0)
    def _():
        pltpu.make_async_copy(wth_hbm, wth_vmem, sem).start()

    for j in range(_BF):
        pooled_j = jnp.dot(pool_ref[...], v_ref[j],
                           preferred_element_type=jnp.float32)
        pooled_ref[pl.ds(i * _BF + j, 1)] = pooled_j[None]
        fm3_ref[pl.ds(i * _BF + j, 1)] = (
            jnp.sum(pooled_j, axis=0) * (1.0 / _TP))[None, None]

    @pl.when(i == _F // _BF - 1)
    def _():
        fm = fm3_ref[:, 0, :]
        mom = jnp.dot(m_ref[...], fm, preferred_element_type=jnp.float32)
        feats = jnp.dot(mom, wagg_ref[...],
                        preferred_element_type=jnp.float32) + bagg_ref[...]
        mu = jnp.mean(feats, axis=-1, keepdims=True)
        var = jnp.mean((feats - mu) ** 2, axis=-1, keepdims=True)
        h = (feats - mu) / jnp.sqrt(var + 1e-5) * lng_ref[...] + lnb_ref[...]
        h = jnp.dot(h, w1_ref[...], preferred_element_type=jnp.float32) + b1_ref[...]
        h = jax.nn.gelu(h)
        z = jnp.dot(h, w2_ref[...], preferred_element_type=jnp.float32) + b2_ref[...]
        g = -jnp.log(-jnp.log(u_ref[...] + _EPS) + _EPS)
        a = (z + g * _LR) / _TEMP
        a0 = a[:, 0:1]
        a1 = a[:, 1:2]
        mx = jnp.maximum(a0, a1)
        e0 = jnp.exp(a0 - mx)
        e1 = jnp.exp(a1 - mx)
        y = e1 / (e0 + e1)
        iota = jax.lax.broadcasted_iota(jnp.int32, (_F, 1), 0)
        ycur = jnp.where(iota < _F - 1, y, -jnp.inf)
        sel = []
        vs = []
        for t in range(_K):
            m = jnp.max(ycur)
            first = jnp.min(jnp.where(ycur == m, iota, jnp.int32(2**30)))
            sel.append(first + 1)
            vs.append((1.0 - m) + m)
            ycur = jnp.where(iota == first, -jnp.inf, ycur)
        s = 1.0
        for v in vs:
            s = s + v
        wcol = jnp.where(iota == 0, 1.0 / s, 0.0)
        for t in range(_K):
            wcol = wcol + jnp.where(iota == sel[t], vs[t] / s, 0.0)
        acc = jnp.zeros((_TP, _D), jnp.float32)
        for c in range(_F // 8):
            chunk = pooled_ref[pl.ds(c * 8, 8), :, :]
            acc = acc + jnp.sum(chunk * wcol[c * 8:(c + 1) * 8, :, None],
                                axis=0)
        pltpu.make_async_copy(wth_hbm, wth_vmem, sem).wait()
        out_ref[...] = jnp.dot(acc, wth_vmem[...],
                               preferred_element_type=jnp.float32) + bth_ref[...]


def kernel(video, U, W_agg, b_agg, ln_g, ln_b, W1, b1, W2, b2, W_th, b_th):
    Mc = jnp.asarray(_M_CONST)
    Pc = jnp.asarray(_P_CONST)
    W2p = jnp.zeros((_D, 128), jnp.float32).at[:, :2].set(W2)
    b2p = jnp.zeros((1, 128), jnp.float32).at[0, :2].set(b2)
    Up = jnp.full((_F, 128), 0.5, jnp.float32).at[: _F - 1, :2].set(U[0])
    out = pl.pallas_call(
        _body,
        grid=(_F // _BF,),
        in_specs=[pl.BlockSpec((_BF, _T, _D), lambda i: (i, 0, 0)),
                  pl.BlockSpec((_TP, _T), lambda i: (0, 0)),
                  pl.BlockSpec((_F, _F), lambda i: (0, 0)),
                  pl.BlockSpec((_D, _D), lambda i: (0, 0)),
                  pl.BlockSpec((1, _D), lambda i: (0, 0)),
                  pl.BlockSpec((1, _D), lambda i: (0, 0)),
                  pl.BlockSpec((1, _D), lambda i: (0, 0)),
                  pl.BlockSpec((_D, _D), lambda i: (0, 0)),
                  pl.BlockSpec((1, _D), lambda i: (0, 0)),
                  pl.BlockSpec((_D, 128), lambda i: (0, 0)),
                  pl.BlockSpec((1, 128), lambda i: (0, 0)),
                  pl.BlockSpec((_F, 128), lambda i: (0, 0)),
                  pl.BlockSpec(memory_space=pl.ANY),
                  pl.BlockSpec((1, _LM), lambda i: (0, 0))],
        out_specs=pl.BlockSpec((_TP, _LM), lambda i: (0, 0)),
        out_shape=jax.ShapeDtypeStruct((_TP, _LM), jnp.float32),
        scratch_shapes=[pltpu.VMEM((_F, _TP, _D), jnp.float32),
                        pltpu.VMEM((_F, 1, _D), jnp.float32),
                        pltpu.VMEM((_D, _LM), jnp.float32),
                        pltpu.SemaphoreType.DMA],
    )(video, Pc, Mc, W_agg, b_agg.reshape(1, _D), ln_g.reshape(1, _D),
      ln_b.reshape(1, _D), W1, b1.reshape(1, _D), W2p, b2p, Up,
      W_th, b_th.reshape(1, _LM))
    return out[None]


# speedup vs baseline: 2.4569x; 1.0005x over previous
import functools

import numpy as np
import jax
import jax.numpy as jnp
from jax.experimental import pallas as pl
from jax.experimental.pallas import tpu as pltpu

_D = 1024
_LM = 4096
_F = 64
_T = 576
_TP = _T // 9
_K = 12
_NSEL = _K + 1
_TEMP = 0.5
_LR = 0.1
_ALPHA = 0.8
_EPS = 1e-20
_BF = 2


def _ema_matrix() -> np.ndarray:
    L = np.zeros((_F - 1, _F - 1), dtype=np.float64)
    for i in range(_F - 1):
        L[i, 0] = (1.0 - _ALPHA) ** i
        for j in range(1, i + 1):
            L[i, j] = _ALPHA * (1.0 - _ALPHA) ** (i - j)
    Dmat = np.zeros((_F - 1, _F), dtype=np.float64)
    for i in range(_F - 1):
        Dmat[i, i] = -1.0
        Dmat[i, i + 1] = 1.0
    M = np.zeros((_F, _F), dtype=np.float64)
    M[: _F - 1] = L @ Dmat
    return M.astype(np.float32)


def _pool_matrix() -> np.ndarray:
    P = np.zeros((_TP, _T), dtype=np.float32)
    for q in range(_TP):
        P[q, 9 * q : 9 * q + 9] = 1.0 / 9.0
    return P


_M_CONST = _ema_matrix()
_P_CONST = _pool_matrix()


def _body(v_ref, pool_ref, m_ref, wagg_hbm, bagg_ref, lng_ref, lnb_ref,
          w1_hbm, b1_ref, w2_ref, b2_ref, u_ref, wth_hbm, bth_ref,
          out_ref, pooled_ref, fm3_ref, wth_vmem, wagg_vmem, w1_vmem,
          sem, sem2, sem3):
    i = pl.program_id(0)

    @pl.when(i == 0)
    def _():
        pltpu.make_async_copy(wth_hbm, wth_vmem, sem).start()
        pltpu.make_async_copy(wagg_hbm, wagg_vmem, sem2).start()
        pltpu.make_async_copy(w1_hbm, w1_vmem, sem3).start()

    for j in range(_BF):
        pooled_j = jnp.dot(pool_ref[...], v_ref[j],
                           preferred_element_type=jnp.float32)
        pooled_ref[pl.ds(i * _BF + j, 1)] = pooled_j[None]
        fm3_ref[pl.ds(i * _BF + j, 1)] = (
            jnp.sum(pooled_j, axis=0) * (1.0 / _TP))[None, None]

    @pl.when(i == _F // _BF - 1)
    def _():
        pltpu.make_async_copy(wagg_hbm, wagg_vmem, sem2).wait()
        pltpu.make_async_copy(w1_hbm, w1_vmem, sem3).wait()
        fm = fm3_ref[:, 0, :]
        mom = jnp.dot(m_ref[...], fm, preferred_element_type=jnp.float32)
        feats = jnp.dot(mom, wagg_vmem[...],
                        preferred_element_type=jnp.float32) + bagg_ref[...]
        mu = jnp.mean(feats, axis=-1, keepdims=True)
        var = jnp.mean((feats - mu) ** 2, axis=-1, keepdims=True)
        h = (feats - mu) / jnp.sqrt(var + 1e-5) * lng_ref[...] + lnb_ref[...]
        h = jnp.dot(h, w1_vmem[...], preferred_element_type=jnp.float32) + b1_ref[...]
        h = jax.nn.gelu(h)
        z = jnp.dot(h, w2_ref[...], preferred_element_type=jnp.float32) + b2_ref[...]
        g = -jnp.log(-jnp.log(u_ref[...] + _EPS) + _EPS)
        a = (z + g * _LR) / _TEMP
        a0 = a[:, 0:1]
        a1 = a[:, 1:2]
        mx = jnp.maximum(a0, a1)
        e0 = jnp.exp(a0 - mx)
        e1 = jnp.exp(a1 - mx)
        y = e1 / (e0 + e1)
        iota = jax.lax.broadcasted_iota(jnp.int32, (_F, 1), 0)
        ycur = jnp.where(iota < _F - 1, y, -jnp.inf)
        sel = []
        vs = []
        for t in range(_K):
            m = jnp.max(ycur)
            first = jnp.min(jnp.where(ycur == m, iota, jnp.int32(2**30)))
            sel.append(first + 1)
            vs.append((1.0 - m) + m)
            ycur = jnp.where(iota == first, -jnp.inf, ycur)
        s = 1.0
        for v in vs:
            s = s + v
        wcol = jnp.where(iota == 0, 1.0 / s, 0.0)
        for t in range(_K):
            wcol = wcol + jnp.where(iota == sel[t], vs[t] / s, 0.0)
        acc = jnp.zeros((_TP, _D), jnp.float32)
        for c in range(_F // 8):
            chunk = pooled_ref[pl.ds(c * 8, 8), :, :]
            acc = acc + jnp.sum(chunk * wcol[c * 8:(c + 1) * 8, :, None],
                                axis=0)
        pltpu.make_async_copy(wth_hbm, wth_vmem, sem).wait()
        out_ref[...] = jnp.dot(acc, wth_vmem[...],
                               preferred_element_type=jnp.float32) + bth_ref[...]


def kernel(video, U, W_agg, b_agg, ln_g, ln_b, W1, b1, W2, b2, W_th, b_th):
    Mc = jnp.asarray(_M_CONST)
    Pc = jnp.asarray(_P_CONST)
    W2p = jnp.zeros((_D, 128), jnp.float32).at[:, :2].set(W2)
    b2p = jnp.zeros((1, 128), jnp.float32).at[0, :2].set(b2)
    Up = jnp.full((_F, 128), 0.5, jnp.float32).at[: _F - 1, :2].set(U[0])
    out = pl.pallas_call(
        _body,
        grid=(_F // _BF,),
        in_specs=[pl.BlockSpec((_BF, _T, _D), lambda i: (i, 0, 0)),
                  pl.BlockSpec((_TP, _T), lambda i: (0, 0)),
                  pl.BlockSpec((_F, _F), lambda i: (0, 0)),
                  pl.BlockSpec(memory_space=pl.ANY),
                  pl.BlockSpec((1, _D), lambda i: (0, 0)),
                  pl.BlockSpec((1, _D), lambda i: (0, 0)),
                  pl.BlockSpec((1, _D), lambda i: (0, 0)),
                  pl.BlockSpec(memory_space=pl.ANY),
                  pl.BlockSpec((1, _D), lambda i: (0, 0)),
                  pl.BlockSpec((_D, 128), lambda i: (0, 0)),
                  pl.BlockSpec((1, 128), lambda i: (0, 0)),
                  pl.BlockSpec((_F, 128), lambda i: (0, 0)),
                  pl.BlockSpec(memory_space=pl.ANY),
                  pl.BlockSpec((1, _LM), lambda i: (0, 0))],
        out_specs=pl.BlockSpec((_TP, _LM), lambda i: (0, 0)),
        out_shape=jax.ShapeDtypeStruct((_TP, _LM), jnp.float32),
        scratch_shapes=[pltpu.VMEM((_F, _TP, _D), jnp.float32),
                        pltpu.VMEM((_F, 1, _D), jnp.float32),
                        pltpu.VMEM((_D, _LM), jnp.float32),
                        pltpu.VMEM((_D, _D), jnp.float32),
                        pltpu.VMEM((_D, _D), jnp.float32),
                        pltpu.SemaphoreType.DMA,
                        pltpu.SemaphoreType.DMA,
                        pltpu.SemaphoreType.DMA],
    )(video, Pc, Mc, W_agg, b_agg.reshape(1, _D), ln_g.reshape(1, _D),
      ln_b.reshape(1, _D), W1, b1.reshape(1, _D), W2p, b2p, Up,
      W_th, b_th.reshape(1, _LM))
    return out[None]
